# Initial kernel scaffold; baseline (speedup 1.0000x reference)
import jax
import jax.numpy as jnp
from jax.experimental import pallas as pl


def kernel(x, tok_emb_weight, pos_emb_weight):
    raise NotImplementedError("write your pallas kernel here")


# speedup vs baseline: 1.0749x; 1.0749x over previous
import functools

import jax
import jax.numpy as jnp
from jax import lax
from jax.experimental import pallas as pl
from jax.experimental.pallas import tpu as pltpu
from jax.experimental.pallas import tpu_sc as plsc

_B = 4
_S = 2048
_D = 128
_NFLAT = _B * _S

_info = plsc.get_sparse_core_info()
_NC = _info.num_cores
_NS = _info.num_subcores
_L = _info.num_lanes
_NW = _NC * _NS
_CHUNK = _NFLAT // _NW
_WPB = _S // _CHUNK
_GSZ = 128


def _emb_body(x_hbm, tok_hbm, pos_hbm, out_hbm, idx_v, tok_v, pos_v, sem):
    wid = lax.axis_index("s") * _NC + lax.axis_index("c")
    base = wid * _CHUNK
    s0 = (wid % _WPB) * _CHUNK

    pltpu.sync_copy(x_hbm.at[pl.ds(base, _CHUNK)], idx_v)
    copies = [
        pltpu.async_copy(
            tok_hbm.at[idx_v.at[pl.ds(j * _GSZ, _GSZ)]],
            tok_v.at[pl.ds(j * _GSZ, _GSZ)],
            sem,
        )
        for j in range(_CHUNK // _GSZ)
    ]
    pltpu.sync_copy(pos_hbm.at[pl.ds(s0, _CHUNK)], pos_v)
    for cp in copies:
        cp.wait()

    def _row(r, carry):
        for j in range(_D // _L):
            sl = pl.ds(j * _L, _L)
            tok_v[r, sl] = tok_v[r, sl] + pos_v[r, sl]
        return carry

    lax.fori_loop(0, _CHUNK, _row, 0)

    pltpu.sync_copy(tok_v, out_hbm.at[pl.ds(base, _CHUNK)])


_emb = functools.partial(
    pl.kernel,
    out_type=jax.ShapeDtypeStruct((_NFLAT, _D), jnp.float32),
    mesh=plsc.VectorSubcoreMesh(core_axis_name="c", subcore_axis_name="s"),
    scratch_types=[
        pltpu.VMEM((_CHUNK,), jnp.int32),
        pltpu.VMEM((_CHUNK, _D), jnp.float32),
        pltpu.VMEM((_CHUNK, _D), jnp.float32),
        pltpu.SemaphoreType.DMA,
    ],
)(_emb_body)


@jax.jit
def kernel(x, tok_emb_weight, pos_emb_weight):
    xf = x.reshape(-1).astype(jnp.int32)
    out = _emb(xf, tok_emb_weight, pos_emb_weight)
    return out.reshape(_B, _S, _D)
